# baseline (speedup 1.0000x reference)
import functools

import jax
import jax.numpy as jnp
from jax import lax
from jax.experimental import pallas as pl
from jax.experimental.pallas import tpu as pltpu
from jax.experimental.pallas import tpu_sc as plsc

N_NODE = 10000
OUT_DIM = 128
N_HEAD = 2

NC = 2
NS = 16
LANES = 16
CGRP = OUT_DIM // LANES
K = 64
NBUF = 4
N_PAD = 10240
ROWS_PER_TILE = N_PAD // NS
WB = 64


def _spmm_sc(tabs, rows4, cols4, vals5):
    n_chunks = rows4.shape[2]

    mesh = plsc.VectorSubcoreMesh(
        core_axis_name="c", subcore_axis_name="s", num_cores=NC, num_subcores=NS)

    @functools.partial(
        pl.kernel,
        mesh=mesh,
        compiler_params=pltpu.CompilerParams(use_tc_tiling_on_sc=False),
        out_type=jax.ShapeDtypeStruct((2 * 3, N_PAD, OUT_DIM), jnp.float32),
        scratch_types=(
            [pltpu.VMEM((n_chunks, K), jnp.int32)]
            + [pltpu.VMEM((K,), jnp.int32)] * NBUF
            + [pltpu.VMEM((K // LANES, LANES), jnp.float32)] * NBUF
            + [pltpu.VMEM((K, OUT_DIM), jnp.float32)] * NBUF
            + [pltpu.VMEM_SHARED((N_PAD, OUT_DIM), jnp.float32)]
            + [pltpu.SemaphoreType.DMA] * (4 * NBUF)
        ),
    )
    def spmm(tabs_hbm, rows_hbm, cols_hbm, vals_hbm, outs_hbm,
             colbuf, *bufs):
        rowb = bufs[0:NBUF]
        valb = bufs[NBUF:2 * NBUF]
        gb = bufs[2 * NBUF:3 * NBUF]
        acc = bufs[3 * NBUF]
        gsem = bufs[3 * NBUF + 1:3 * NBUF + 1 + NBUF]
        rsem = bufs[3 * NBUF + 1 + NBUF:3 * NBUF + 1 + 2 * NBUF]
        vsem = bufs[3 * NBUF + 1 + 2 * NBUF:3 * NBUF + 1 + 3 * NBUF]
        ssem = bufs[3 * NBUF + 1 + 3 * NBUF:3 * NBUF + 1 + 4 * NBUF]
        c = lax.axis_index("c")
        s = lax.axis_index("s")
        row0 = s * ROWS_PER_TILE
        zero16 = jnp.zeros((LANES,), jnp.float32)

        pltpu.sync_copy(cols_hbm.at[c, s], colbuf)

        def one_pass(tab_hbm, out_hbm):
            for j0 in range(NBUF - 1):
                pltpu.async_copy(vals_hbm.at[c, s, j0], valb[j0], vsem[j0])
                pltpu.async_copy(rows_hbm.at[c, s, j0], rowb[j0], rsem[j0])
                pltpu.async_copy(
                    tab_hbm.at[colbuf.at[j0]], gb[j0], gsem[j0])

            def zrow(r, _):
                for g in range(CGRP):
                    gb[NBUF - 1][r, pl.ds(g * LANES, LANES)] = zero16
                return 0

            lax.fori_loop(0, WB, zrow, 0)
            for j in range(ROWS_PER_TILE // WB):
                pltpu.sync_copy(gb[NBUF - 1], acc.at[pl.ds(row0 + j * WB, WB)])
            plsc.subcore_barrier()

            def quad(p, _):
                for b in range(NBUF):
                    j = NBUF * p + b

                    @pl.when(j + NBUF - 1 < n_chunks)
                    def _():
                        @pl.when(j >= 1)
                        def _():
                            pltpu.make_async_copy(
                                gb[b - 1 if b else NBUF - 1],
                                acc.at[rowb[b - 1 if b else NBUF - 1]],
                                ssem[b - 1 if b else NBUF - 1]).wait()
                        jn = j + NBUF - 1
                        bn = (b + NBUF - 1) % NBUF
                        pltpu.async_copy(vals_hbm.at[c, s, jn], valb[bn], vsem[bn])
                        pltpu.async_copy(rows_hbm.at[c, s, jn], rowb[bn], rsem[bn])
                        pltpu.async_copy(
                            tab_hbm.at[colbuf.at[jn]], gb[bn], gsem[bn])

                    pltpu.make_async_copy(
                        tab_hbm.at[colbuf.at[j]], gb[b], gsem[b]).wait()
                    pltpu.make_async_copy(
                        vals_hbm.at[c, s, j], valb[b], vsem[b]).wait()

                    @plsc.parallel_loop(0, K // LANES, unroll=1)
                    def _(q):
                        vvec = valb[b][q]
                        for i in range(LANES):
                            vv = jnp.broadcast_to(vvec[i], (LANES,))
                            e = q * LANES + i
                            for g in range(CGRP):
                                sl = pl.ds(g * LANES, LANES)
                                gb[b][e, sl] = gb[b][e, sl] * vv

                    pltpu.make_async_copy(
                        rows_hbm.at[c, s, j], rowb[b], rsem[b]).wait()
                    pltpu.async_copy(
                        gb[b], acc.at[rowb[b]], ssem[b], add=True)
                return 0

            lax.fori_loop(0, n_chunks // NBUF, quad, 0)
            for b in range(NBUF):
                pltpu.make_async_copy(gb[b], acc.at[rowb[b]], ssem[b]).wait()
            plsc.subcore_barrier()

            for j in range(ROWS_PER_TILE // WB):
                r0 = row0 + j * WB
                bounce = gb[j % NBUF]
                pltpu.sync_copy(acc.at[pl.ds(r0, WB)], bounce)
                pltpu.sync_copy(bounce, out_hbm.at[pl.ds(r0, WB)])

        def pass_body(pidx, _):
            t = lax.rem(pidx + c, 3)
            one_pass(tabs_hbm.at[t], outs_hbm.at[3 * c + t])
            return 0

        lax.fori_loop(0, 3, pass_body, 0)

    return spmm(tabs, rows4, cols4, vals5)


def _gru_tc(apart, weight_vars, U_r, U_u, U_h, b_r, b_u, b_h):
    R = 1000
    nb = N_NODE // R

    def body(ab, wv, ur, uu, uh, br, bu, bh, o):
        prev = wv[0]
        a_wr = ab[0] + ab[3]
        a_wu = ab[1] + ab[4]
        a_wh = ab[2] + ab[5]
        f32 = jnp.float32
        reset = jax.nn.sigmoid(
            a_wr + jnp.dot(prev, ur[:], preferred_element_type=f32) + br[:])
        update = jax.nn.sigmoid(
            a_wu + jnp.dot(prev, uu[:], preferred_element_type=f32) + bu[:])
        h_cap = jnp.tanh(
            a_wh + jnp.dot(reset * prev, uh[:], preferred_element_type=f32) + bh[:])
        o[0] = (1.0 - update) * prev + update * h_cap

    b_spec = pl.BlockSpec((R, OUT_DIM), lambda h, i: (i, 0))
    u_spec = pl.BlockSpec((OUT_DIM, OUT_DIM), lambda h, i: (0, 0))
    return pl.pallas_call(
        body,
        grid=(N_HEAD, nb),
        in_specs=[
            pl.BlockSpec((6, R, OUT_DIM), lambda h, i: (0, i, 0)),
            pl.BlockSpec((1, R, OUT_DIM), lambda h, i: (h, i, 0)),
            u_spec, u_spec, u_spec,
            b_spec, b_spec, b_spec,
        ],
        out_specs=pl.BlockSpec((1, R, OUT_DIM), lambda h, i: (h, i, 0)),
        out_shape=jax.ShapeDtypeStruct((N_HEAD, N_NODE, OUT_DIM), jnp.float32),
    )(apart, weight_vars, U_r, U_u, U_h, b_r, b_u, b_h)


def kernel(edge_index, adj_values, weight_vars,
           W_r, U_r, b_r, W_u, U_u, b_u, W_h, U_h, b_h):
    rows = edge_index[0]
    cols = edge_index[1]
    n_edge = rows.shape[0]
    grain = NC * NS * K * NBUF
    e_pad = ((n_edge + grain - 1) // grain) * grain
    pad = e_pad - n_edge
    if pad:
        spread = jnp.arange(pad, dtype=jnp.int32) % N_NODE
        rows = jnp.concatenate([rows, spread])
        cols = jnp.concatenate([cols, spread])
        adj_values = jnp.pad(adj_values, (0, pad))

    n_chunks = e_pad // (NC * NS * K)
    rows4 = rows.reshape(NC, NS, n_chunks, K)
    cols4 = cols.reshape(NC, NS, n_chunks, K)
    vals5 = adj_values.reshape(NC, NS, n_chunks, K // LANES, LANES)

    apart = _spmm_sc(jnp.stack([W_r, W_u, W_h]), rows4, cols4, vals5)
    return _gru_tc(apart, weight_vars, U_r, U_u, U_h, b_r, b_u, b_h)
